# baseline (speedup 1.0000x reference)
import functools

import jax
import jax.numpy as jnp
from jax import lax
from jax.experimental import pallas as pl
from jax.experimental.pallas import tpu as pltpu
from jax.experimental.pallas import tpu_sc as plsc

N = 10000
E = 320000
F_IN = 128
H = 256
C = 32
G = 64

_K = 80
_NPAD = 10112
_ROWS_PER_TILE = _NPAD // 16


def _sc_segment_sum(table, idx, zeros, nchunks):
    mesh = plsc.VectorSubcoreMesh(
        core_axis_name="c", subcore_axis_name="s", num_cores=2, num_subcores=16)

    @functools.partial(
        pl.kernel,
        out_type=jax.ShapeDtypeStruct((2 * _NPAD, 128), jnp.float32),
        mesh=mesh,
        scratch_types=[
            pltpu.VMEM((4, 2, _K), jnp.int32),
            pltpu.VMEM((4, _K, 128), jnp.float32),
            pltpu.VMEM_SHARED((_NPAD, 128), jnp.float32),
            pltpu.SemaphoreType.DMA,
            pltpu.SemaphoreType.DMA,
            pltpu.SemaphoreType.DMA,
        ],
    )
    def k(table_h, idx_h, zeros_h, out_h, idx_v, rows_v, acc,
          gsem, isem, ssem):
        cid = lax.axis_index("c")
        sid = lax.axis_index("s")
        wid = cid * 16 + sid
        pltpu.sync_copy(zeros_h, acc.at[pl.ds(sid * _ROWS_PER_TILE, _ROWS_PER_TILE)])
        plsc.subcore_barrier()

        def idx_load(chunk, slot):
            pltpu.async_copy(idx_h.at[wid, chunk], idx_v.at[slot], isem)

        def idx_wait(chunk, slot):
            pltpu.make_async_copy(idx_h.at[wid, chunk], idx_v.at[slot],
                                  isem).wait()

        def gather_start(chunk, slot):
            pltpu.async_copy(table_h.at[idx_v.at[slot, 0]], rows_v.at[slot],
                             gsem)

        def gather_wait(slot):
            pltpu.make_async_copy(table_h.at[idx_v.at[slot, 0]],
                                  rows_v.at[slot], gsem).wait()

        def scatter_start(slot):
            pltpu.async_copy(rows_v.at[slot], acc.at[idx_v.at[slot, 1]], ssem,
                             add=True)

        def scatter_wait(slot):
            pltpu.make_async_copy(rows_v.at[slot], acc.at[idx_v.at[slot, 1]],
                                  ssem).wait()

        idx_load(0, 0)
        idx_load(jnp.minimum(1, nchunks - 1), 1)
        idx_wait(0, 0)
        gather_start(0, 0)

        def body(j, carry):
            s0 = j % 4
            s1 = (j + 1) % 4
            s2 = (j + 2) % 4
            jn = jnp.minimum(j + 1, nchunks - 1)
            jn2 = jnp.minimum(j + 2, nchunks - 1)

            @pl.when(j > 1)
            def _():
                scatter_wait(s2)

            idx_load(jn2, s2)
            idx_wait(jn, s1)
            gather_wait(s0)
            gather_start(jn, s1)
            scatter_start(s0)
            return carry

        lax.fori_loop(0, nchunks, body, 0)
        lc = nchunks - 1
        scatter_wait((nchunks - 2) % 4)
        scatter_wait(lc % 4)
        gather_wait(nchunks % 4)
        idx_wait(lc, (nchunks + 1) % 4)
        plsc.subcore_barrier()
        base = sid * _ROWS_PER_TILE
        pltpu.sync_copy(
            acc.at[pl.ds(base, _ROWS_PER_TILE)],
            out_h.at[pl.ds(cid * _NPAD + base, _ROWS_PER_TILE)],
        )

    return k(table, idx, zeros)


def _tc_conv0bn(x, aggP, w1, b1, w2, b2, g, b):
    nb = 10
    bn = N // nb

    def body(x_ref, agg_ref, w1_ref, b1_ref, w2_ref, b2_ref, g_ref, b_ref,
             out_ref, vbuf, st_ref):
        i = pl.program_id(0)

        @pl.when(i < nb)
        def _():
            s = x_ref[...] + agg_ref[0] + agg_ref[1]
            t = jnp.tanh(jnp.dot(s, w1_ref[...],
                                 preferred_element_type=jnp.float32)
                         + b1_ref[...])
            u = (jnp.dot(t, w2_ref[...], preferred_element_type=jnp.float32)
                 + b2_ref[...])
            v = jnp.tanh(u)
            vbuf[pl.ds(i * bn, bn), :] = v
            st = jnp.stack([jnp.sum(v, axis=0), jnp.sum(v * v, axis=0)])

            @pl.when(i == 0)
            def _():
                st_ref[...] = st

            @pl.when(i > 0)
            def _():
                st_ref[...] = st_ref[...] + st

        @pl.when(i == nb)
        def _():
            m = st_ref[0] / float(N)
            var = st_ref[1] / float(N) - m * m
            a = g_ref[...] * lax.rsqrt(var + 1e-5)
            c = b_ref[...] - m * a
            av = vbuf[...] * a + c
            out_ref[0] = av[:, :128]
            out_ref[1] = av[:, 128:]

    clam = lambda i: (jnp.minimum(i, nb - 1), 0)
    clam3 = lambda i: (0, jnp.minimum(i, nb - 1), 0)
    return pl.pallas_call(
        body,
        grid=(nb + 1,),
        in_specs=[
            pl.BlockSpec((bn, F_IN), clam),
            pl.BlockSpec((2, bn, 128), clam3),
            pl.BlockSpec((F_IN, H), lambda i: (0, 0)),
            pl.BlockSpec((1, H), lambda i: (0, 0)),
            pl.BlockSpec((H, H), lambda i: (0, 0)),
            pl.BlockSpec((1, H), lambda i: (0, 0)),
            pl.BlockSpec((1, H), lambda i: (0, 0)),
            pl.BlockSpec((1, H), lambda i: (0, 0)),
        ],
        out_specs=pl.BlockSpec((2, N, 128), lambda i: (0, 0, 0)),
        out_shape=jax.ShapeDtypeStruct((2, N, 128), jnp.float32),
        scratch_shapes=[
            pltpu.VMEM((N, H), jnp.float32),
            pltpu.VMEM((2, H), jnp.float32),
        ],
    )(x, aggP, w1, b1, w2, b2, g, b)


def _tc_final(h0S, agg1S, batch3, w1, b1, w2, b2, g, bb, l1w, l1b, l2w, l2b):
    nb = 10
    bn = N // nb

    def body(h_ref, agg_ref, bt_ref, w1_ref, b1_ref, w2_ref, b2_ref, g_ref,
             bb_ref, l1w_ref, l1b_ref, l2w_ref, l2b_ref, o_ref,
             pooled, cnt, st):
        i = pl.program_id(0)

        @pl.when(i == 0)
        def _():
            pooled[...] = jnp.zeros((G, H), jnp.float32)
            cnt[...] = jnp.zeros((1, G), jnp.float32)
            st[...] = jnp.zeros((2, H), jnp.float32)

        s = jnp.concatenate(
            [h_ref[0] + agg_ref[0], h_ref[1] + agg_ref[1]], axis=1)
        t = jnp.tanh(jnp.dot(s, w1_ref[...], preferred_element_type=jnp.float32)
                     + b1_ref[...])
        u = jnp.dot(t, w2_ref[...], preferred_element_type=jnp.float32) + b2_ref[...]
        v = jnp.tanh(u)

        gids = bt_ref[0, 0]
        oh = (gids[:, None] ==
              lax.broadcasted_iota(jnp.int32, (bn, G), 1)).astype(jnp.float32)
        pooled[...] = pooled[...] + lax.dot_general(
            oh, v, (((0,), (0,)), ((), ())), preferred_element_type=jnp.float32)
        cnt[...] = cnt[...] + jnp.sum(oh, axis=0, keepdims=True)
        st[...] = st[...] + jnp.stack([jnp.sum(v, axis=0), jnp.sum(v * v, axis=0)])

        @pl.when(i == nb - 1)
        def _():
            m = st[0] / float(N)
            var = st[1] / float(N) - m * m
            a = g_ref[...] * lax.rsqrt(var + 1e-5)
            c = bb_ref[...] - m * a
            cc = cnt[...].reshape(G, 1)
            pm = pooled[...] / jnp.maximum(cc, 1.0)
            pb = jnp.where(cc > 0.0, pm * a + c, 0.0)
            o = jnp.dot(jnp.tanh(jnp.dot(pb, l1w_ref[...],
                                         preferred_element_type=jnp.float32)
                                 + l1b_ref[...]),
                        l2w_ref[...], preferred_element_type=jnp.float32)
            o_ref[...] = o + l2b_ref[...]

    return pl.pallas_call(
        body,
        grid=(nb,),
        in_specs=[
            pl.BlockSpec((2, bn, 128), lambda i: (0, i, 0)),
            pl.BlockSpec((2, bn, 128), lambda i: (0, i, 0)),
            pl.BlockSpec((1, 1, bn), lambda i: (i, 0, 0)),
            pl.BlockSpec((H, H), lambda i: (0, 0)),
            pl.BlockSpec((1, H), lambda i: (0, 0)),
            pl.BlockSpec((H, H), lambda i: (0, 0)),
            pl.BlockSpec((1, H), lambda i: (0, 0)),
            pl.BlockSpec((1, H), lambda i: (0, 0)),
            pl.BlockSpec((1, H), lambda i: (0, 0)),
            pl.BlockSpec((H, H), lambda i: (0, 0)),
            pl.BlockSpec((1, H), lambda i: (0, 0)),
            pl.BlockSpec((H, C), lambda i: (0, 0)),
            pl.BlockSpec((1, C), lambda i: (0, 0)),
        ],
        out_specs=pl.BlockSpec((G, C), lambda i: (0, 0)),
        out_shape=jax.ShapeDtypeStruct((G, C), jnp.float32),
        scratch_shapes=[
            pltpu.VMEM((G, H), jnp.float32),
            pltpu.VMEM((1, G), jnp.float32),
            pltpu.VMEM((2, H), jnp.float32),
        ],
    )(h0S, agg1S, batch3, w1, b1, w2, b2, g, bb, l1w, l1b, l2w, l2b)


def kernel(x, edge_index, batch, conv0_w1, conv0_b1, conv0_w2, conv0_b2,
           bn0_g, bn0_b, conv1_w1, conv1_b1, conv1_w2, conv1_b2, bn1_g, bn1_b,
           lin1_w, lin1_b, lin2_w, lin2_b):
    src = edge_index[0].astype(jnp.int32)
    dst = edge_index[1].astype(jnp.int32)
    zeros = jnp.zeros((_ROWS_PER_TILE, 128), jnp.float32)

    nch0 = E // (32 * _K)
    idxA = jnp.stack([src.reshape(32, nch0, _K), dst.reshape(32, nch0, _K)],
                     axis=2)
    agg0P = _sc_segment_sum(x, idxA, zeros, nch0)
    agg0P = agg0P.reshape(2, _NPAD, 128)

    h0S = _tc_conv0bn(
        x, agg0P, conv0_w1, conv0_b1.reshape(1, H), conv0_w2,
        conv0_b2.reshape(1, H), bn0_g.reshape(1, H), bn0_b.reshape(1, H))

    nch1 = E // (16 * _K)
    s3 = src.reshape(1, 16, nch1, _K)
    off = (jnp.arange(2, dtype=jnp.int32) * N).reshape(2, 1, 1, 1)
    srcB = (s3 + off).reshape(32, nch1, _K)
    dstB = jnp.broadcast_to(
        dst.reshape(1, 16, nch1, _K), (2, 16, nch1, _K)).reshape(32, nch1, _K)
    idxB = jnp.stack([srcB, dstB], axis=2)
    agg1S = _sc_segment_sum(h0S.reshape(2 * N, 128), idxB, zeros, nch1)
    agg1S = agg1S.reshape(2, _NPAD, 128)

    o = _tc_final(
        h0S, agg1S, batch.astype(jnp.int32).reshape(10, 1, N // 10),
        conv1_w1, conv1_b1.reshape(1, H), conv1_w2, conv1_b2.reshape(1, H),
        bn1_g.reshape(1, H), bn1_b.reshape(1, H),
        lin1_w, lin1_b.reshape(1, H), lin2_w, lin2_b.reshape(1, C))
    return o


# speedup vs baseline: 1.2095x; 1.2095x over previous
import functools

import jax
import jax.numpy as jnp
from jax import lax
from jax.experimental import pallas as pl
from jax.experimental.pallas import tpu as pltpu
from jax.experimental.pallas import tpu_sc as plsc

N = 10000
E = 320000
F_IN = 128
H = 256
C = 32
G = 64

_K = 125
_NPAD = 10112
_ROWS_PER_TILE = _NPAD // 16


def _sc_segment_sum(table, idx, zeros, nchunks):
    mesh = plsc.VectorSubcoreMesh(
        core_axis_name="c", subcore_axis_name="s", num_cores=2, num_subcores=16)

    @functools.partial(
        pl.kernel,
        out_type=jax.ShapeDtypeStruct((2 * _NPAD, 128), jnp.float32),
        mesh=mesh,
        scratch_types=[
            pltpu.VMEM((4, 2, _K), jnp.int32),
            pltpu.VMEM((3, _K, 128), jnp.float32),
            pltpu.VMEM_SHARED((_NPAD, 128), jnp.float32),
            pltpu.SemaphoreType.DMA,
            pltpu.SemaphoreType.DMA,
            pltpu.SemaphoreType.DMA,
        ],
    )
    def k(table_h, idx_h, zeros_h, out_h, idx_v, rows_v, acc,
          gsem, isem, ssem):
        cid = lax.axis_index("c")
        sid = lax.axis_index("s")
        wid = cid * 16 + sid
        pltpu.sync_copy(zeros_h, acc.at[pl.ds(sid * _ROWS_PER_TILE, _ROWS_PER_TILE)])
        plsc.subcore_barrier()

        def idx_load(chunk, islot):
            pltpu.async_copy(idx_h.at[wid, chunk], idx_v.at[islot], isem)

        def idx_wait(chunk, islot):
            pltpu.make_async_copy(idx_h.at[wid, chunk], idx_v.at[islot],
                                  isem).wait()

        def gather_start(islot, rslot):
            pltpu.async_copy(table_h.at[idx_v.at[islot, 0]], rows_v.at[rslot],
                             gsem)

        def gather_wait(islot, rslot):
            pltpu.make_async_copy(table_h.at[idx_v.at[islot, 0]],
                                  rows_v.at[rslot], gsem).wait()

        def scatter_start(islot, rslot):
            pltpu.async_copy(rows_v.at[rslot], acc.at[idx_v.at[islot, 1]],
                             ssem, add=True)

        def scatter_wait(islot, rslot):
            pltpu.make_async_copy(rows_v.at[rslot], acc.at[idx_v.at[islot, 1]],
                                  ssem).wait()

        idx_load(0, 0)
        idx_load(jnp.minimum(1, nchunks - 1), 1)
        idx_wait(0, 0)
        gather_start(0, 0)

        def body(j, carry):
            jn = jnp.minimum(j + 1, nchunks - 1)
            jn2 = jnp.minimum(j + 2, nchunks - 1)

            @pl.when(j > 1)
            def _():
                scatter_wait((j + 2) % 4, (j + 1) % 3)

            idx_load(jn2, (j + 2) % 4)
            idx_wait(jn, (j + 1) % 4)
            gather_wait(j % 4, j % 3)
            gather_start((j + 1) % 4, (j + 1) % 3)
            scatter_start(j % 4, j % 3)
            return carry

        lax.fori_loop(0, nchunks, body, 0)
        nc = nchunks
        scatter_wait((nc - 2) % 4, (nc - 2) % 3)
        scatter_wait((nc - 1) % 4, (nc - 1) % 3)
        gather_wait(nc % 4, nc % 3)
        idx_wait(nc - 1, (nc + 1) % 4)
        plsc.subcore_barrier()
        base = sid * _ROWS_PER_TILE
        pltpu.sync_copy(
            acc.at[pl.ds(base, _ROWS_PER_TILE)],
            out_h.at[pl.ds(cid * _NPAD + base, _ROWS_PER_TILE)],
        )

    return k(table, idx, zeros)


def _tc_conv0bn(x, aggP, w1, b1, w2, b2, g, b):
    nb = 10
    bn = N // nb

    def body(x_ref, agg_ref, w1_ref, b1_ref, w2_ref, b2_ref, g_ref, b_ref,
             out_ref, vbuf, st_ref):
        i = pl.program_id(0)

        @pl.when(i < nb)
        def _():
            s = x_ref[...] + agg_ref[0] + agg_ref[1]
            t = jnp.tanh(jnp.dot(s, w1_ref[...],
                                 preferred_element_type=jnp.float32)
                         + b1_ref[...])
            u = (jnp.dot(t, w2_ref[...], preferred_element_type=jnp.float32)
                 + b2_ref[...])
            v = jnp.tanh(u)
            vbuf[pl.ds(i * bn, bn), :] = v
            st = jnp.stack([jnp.sum(v, axis=0), jnp.sum(v * v, axis=0)])

            @pl.when(i == 0)
            def _():
                st_ref[...] = st

            @pl.when(i > 0)
            def _():
                st_ref[...] = st_ref[...] + st

        @pl.when(i == nb)
        def _():
            m = st_ref[0] / float(N)
            var = st_ref[1] / float(N) - m * m
            a = g_ref[...] * lax.rsqrt(var + 1e-5)
            c = b_ref[...] - m * a
            av = vbuf[...] * a + c
            out_ref[0] = av[:, :128]
            out_ref[1] = av[:, 128:]

    clam = lambda i: (jnp.minimum(i, nb - 1), 0)
    clam3 = lambda i: (0, jnp.minimum(i, nb - 1), 0)
    return pl.pallas_call(
        body,
        grid=(nb + 1,),
        in_specs=[
            pl.BlockSpec((bn, F_IN), clam),
            pl.BlockSpec((2, bn, 128), clam3),
            pl.BlockSpec((F_IN, H), lambda i: (0, 0)),
            pl.BlockSpec((1, H), lambda i: (0, 0)),
            pl.BlockSpec((H, H), lambda i: (0, 0)),
            pl.BlockSpec((1, H), lambda i: (0, 0)),
            pl.BlockSpec((1, H), lambda i: (0, 0)),
            pl.BlockSpec((1, H), lambda i: (0, 0)),
        ],
        out_specs=pl.BlockSpec((2, N, 128), lambda i: (0, 0, 0)),
        out_shape=jax.ShapeDtypeStruct((2, N, 128), jnp.float32),
        scratch_shapes=[
            pltpu.VMEM((N, H), jnp.float32),
            pltpu.VMEM((2, H), jnp.float32),
        ],
    )(x, aggP, w1, b1, w2, b2, g, b)


def _tc_final(h0S, agg1S, batch3, w1, b1, w2, b2, g, bb, l1w, l1b, l2w, l2b):
    nb = 10
    bn = N // nb

    def body(h_ref, agg_ref, bt_ref, w1_ref, b1_ref, w2_ref, b2_ref, g_ref,
             bb_ref, l1w_ref, l1b_ref, l2w_ref, l2b_ref, o_ref,
             pooled, cnt, st):
        i = pl.program_id(0)

        @pl.when(i == 0)
        def _():
            pooled[...] = jnp.zeros((G, H), jnp.float32)
            cnt[...] = jnp.zeros((1, G), jnp.float32)
            st[...] = jnp.zeros((2, H), jnp.float32)

        s = jnp.concatenate(
            [h_ref[0] + agg_ref[0], h_ref[1] + agg_ref[1]], axis=1)
        t = jnp.tanh(jnp.dot(s, w1_ref[...], preferred_element_type=jnp.float32)
                     + b1_ref[...])
        u = jnp.dot(t, w2_ref[...], preferred_element_type=jnp.float32) + b2_ref[...]
        v = jnp.tanh(u)

        gids = bt_ref[0, 0]
        oh = (gids[:, None] ==
              lax.broadcasted_iota(jnp.int32, (bn, G), 1)).astype(jnp.float32)
        pooled[...] = pooled[...] + lax.dot_general(
            oh, v, (((0,), (0,)), ((), ())), preferred_element_type=jnp.float32)
        cnt[...] = cnt[...] + jnp.sum(oh, axis=0, keepdims=True)
        st[...] = st[...] + jnp.stack([jnp.sum(v, axis=0), jnp.sum(v * v, axis=0)])

        @pl.when(i == nb - 1)
        def _():
            m = st[0] / float(N)
            var = st[1] / float(N) - m * m
            a = g_ref[...] * lax.rsqrt(var + 1e-5)
            c = bb_ref[...] - m * a
            cc = cnt[...].reshape(G, 1)
            pm = pooled[...] / jnp.maximum(cc, 1.0)
            pb = jnp.where(cc > 0.0, pm * a + c, 0.0)
            o = jnp.dot(jnp.tanh(jnp.dot(pb, l1w_ref[...],
                                         preferred_element_type=jnp.float32)
                                 + l1b_ref[...]),
                        l2w_ref[...], preferred_element_type=jnp.float32)
            o_ref[...] = o + l2b_ref[...]

    return pl.pallas_call(
        body,
        grid=(nb,),
        in_specs=[
            pl.BlockSpec((2, bn, 128), lambda i: (0, i, 0)),
            pl.BlockSpec((2, bn, 128), lambda i: (0, i, 0)),
            pl.BlockSpec((1, 1, bn), lambda i: (i, 0, 0)),
            pl.BlockSpec((H, H), lambda i: (0, 0)),
            pl.BlockSpec((1, H), lambda i: (0, 0)),
            pl.BlockSpec((H, H), lambda i: (0, 0)),
            pl.BlockSpec((1, H), lambda i: (0, 0)),
            pl.BlockSpec((1, H), lambda i: (0, 0)),
            pl.BlockSpec((1, H), lambda i: (0, 0)),
            pl.BlockSpec((H, H), lambda i: (0, 0)),
            pl.BlockSpec((1, H), lambda i: (0, 0)),
            pl.BlockSpec((H, C), lambda i: (0, 0)),
            pl.BlockSpec((1, C), lambda i: (0, 0)),
        ],
        out_specs=pl.BlockSpec((G, C), lambda i: (0, 0)),
        out_shape=jax.ShapeDtypeStruct((G, C), jnp.float32),
        scratch_shapes=[
            pltpu.VMEM((G, H), jnp.float32),
            pltpu.VMEM((1, G), jnp.float32),
            pltpu.VMEM((2, H), jnp.float32),
        ],
    )(h0S, agg1S, batch3, w1, b1, w2, b2, g, bb, l1w, l1b, l2w, l2b)


def kernel(x, edge_index, batch, conv0_w1, conv0_b1, conv0_w2, conv0_b2,
           bn0_g, bn0_b, conv1_w1, conv1_b1, conv1_w2, conv1_b2, bn1_g, bn1_b,
           lin1_w, lin1_b, lin2_w, lin2_b):
    src = edge_index[0].astype(jnp.int32)
    dst = edge_index[1].astype(jnp.int32)
    zeros = jnp.zeros((_ROWS_PER_TILE, 128), jnp.float32)

    nch0 = E // (32 * _K)
    idxA = jnp.stack([src.reshape(32, nch0, _K), dst.reshape(32, nch0, _K)],
                     axis=2)
    agg0P = _sc_segment_sum(x, idxA, zeros, nch0)
    agg0P = agg0P.reshape(2, _NPAD, 128)

    h0S = _tc_conv0bn(
        x, agg0P, conv0_w1, conv0_b1.reshape(1, H), conv0_w2,
        conv0_b2.reshape(1, H), bn0_g.reshape(1, H), bn0_b.reshape(1, H))

    nch1 = E // (16 * _K)
    s3 = src.reshape(1, 16, nch1, _K)
    off = (jnp.arange(2, dtype=jnp.int32) * N).reshape(2, 1, 1, 1)
    srcB = (s3 + off).reshape(32, nch1, _K)
    dstB = jnp.broadcast_to(
        dst.reshape(1, 16, nch1, _K), (2, 16, nch1, _K)).reshape(32, nch1, _K)
    idxB = jnp.stack([srcB, dstB], axis=2)
    agg1S = _sc_segment_sum(h0S.reshape(2 * N, 128), idxB, zeros, nch1)
    agg1S = agg1S.reshape(2, _NPAD, 128)

    o = _tc_final(
        h0S, agg1S, batch.astype(jnp.int32).reshape(10, 1, N // 10),
        conv1_w1, conv1_b1.reshape(1, H), conv1_w2, conv1_b2.reshape(1, H),
        bn1_g.reshape(1, H), bn1_b.reshape(1, H),
        lin1_w, lin1_b.reshape(1, H), lin2_w, lin2_b.reshape(1, C))
    return o


# speedup vs baseline: 1.4204x; 1.1743x over previous
import functools

import jax
import jax.numpy as jnp
from jax import lax
from jax.experimental import pallas as pl
from jax.experimental.pallas import tpu as pltpu
from jax.experimental.pallas import tpu_sc as plsc

N = 10000
E = 320000
F_IN = 128
H = 256
C = 32
G = 64

_K = 80
_NPAD = 10112
_ROWS_PER_TILE = _NPAD // 16


def _sc_segment_sum(table, idx, zeros, nchunks):
    mesh = plsc.VectorSubcoreMesh(
        core_axis_name="c", subcore_axis_name="s", num_cores=2, num_subcores=16)

    @functools.partial(
        pl.kernel,
        out_type=jax.ShapeDtypeStruct((2 * _NPAD, 128), jnp.float32),
        mesh=mesh,
        scratch_types=[
            pltpu.VMEM((6, 2, _K), jnp.int32),
            pltpu.VMEM((4, _K, 128), jnp.float32),
            pltpu.VMEM_SHARED((_NPAD, 128), jnp.float32),
            pltpu.SemaphoreType.DMA,
            pltpu.SemaphoreType.DMA,
            pltpu.SemaphoreType.DMA,
        ],
    )
    def k(table_h, idx_h, zeros_h, out_h, idx_v, rows_v, acc,
          gsem, isem, ssem):
        cid = lax.axis_index("c")
        sid = lax.axis_index("s")
        wid = cid * 16 + sid
        pltpu.sync_copy(zeros_h, acc.at[pl.ds(sid * _ROWS_PER_TILE, _ROWS_PER_TILE)])
        plsc.subcore_barrier()

        def idx_load(chunk, islot):
            pltpu.async_copy(idx_h.at[wid, chunk], idx_v.at[islot], isem)

        def idx_wait(chunk, islot):
            pltpu.make_async_copy(idx_h.at[wid, chunk], idx_v.at[islot],
                                  isem).wait()

        def gather_start(islot, rslot):
            pltpu.async_copy(table_h.at[idx_v.at[islot, 0]], rows_v.at[rslot],
                             gsem)

        def gather_wait(islot, rslot):
            pltpu.make_async_copy(table_h.at[idx_v.at[islot, 0]],
                                  rows_v.at[rslot], gsem).wait()

        def scatter_start(islot, rslot):
            pltpu.async_copy(rows_v.at[rslot], acc.at[idx_v.at[islot, 1]],
                             ssem, add=True)

        def scatter_wait(islot, rslot):
            pltpu.make_async_copy(rows_v.at[rslot], acc.at[idx_v.at[islot, 1]],
                                  ssem).wait()

        idx_load(0, 0)
        idx_load(jnp.minimum(1, nchunks - 1), 1)
        idx_load(jnp.minimum(2, nchunks - 1), 2)
        idx_wait(0, 0)
        gather_start(0, 0)
        idx_wait(jnp.minimum(1, nchunks - 1), 1)
        gather_start(1, 1)

        def body(j, carry):
            jn2 = jnp.minimum(j + 2, nchunks - 1)
            jn3 = jnp.minimum(j + 3, nchunks - 1)

            @pl.when(j > 1)
            def _():
                scatter_wait((j - 2) % 6, (j + 2) % 4)

            idx_load(jn3, (j + 3) % 6)
            idx_wait(jn2, (j + 2) % 6)
            gather_wait(j % 6, j % 4)
            gather_start((j + 2) % 6, (j + 2) % 4)
            scatter_start(j % 6, j % 4)
            return carry

        lax.fori_loop(0, nchunks, body, 0)
        nc = nchunks
        scatter_wait((nc - 2) % 6, (nc - 2) % 4)
        scatter_wait((nc - 1) % 6, (nc - 1) % 4)
        gather_wait(nc % 6, nc % 4)
        gather_wait((nc + 1) % 6, (nc + 1) % 4)
        idx_wait(nc - 1, (nc + 2) % 6)
        plsc.subcore_barrier()
        base = sid * _ROWS_PER_TILE
        pltpu.sync_copy(
            acc.at[pl.ds(base, _ROWS_PER_TILE)],
            out_h.at[pl.ds(cid * _NPAD + base, _ROWS_PER_TILE)],
        )

    return k(table, idx, zeros)


def _tc_conv0bn(x, aggP, w1, b1, w2, b2, g, b):
    nb = 10
    bn = N // nb

    def body(x_ref, agg_ref, w1_ref, b1_ref, w2_ref, b2_ref, g_ref, b_ref,
             out_ref, vbuf, st_ref):
        i = pl.program_id(0)

        @pl.when(i < nb)
        def _():
            s = x_ref[...] + agg_ref[0] + agg_ref[1]
            t = jnp.tanh(jnp.dot(s, w1_ref[...],
                                 preferred_element_type=jnp.float32)
                         + b1_ref[...])
            u = (jnp.dot(t, w2_ref[...], preferred_element_type=jnp.float32)
                 + b2_ref[...])
            v = jnp.tanh(u)
            vbuf[pl.ds(i * bn, bn), :] = v
            st = jnp.stack([jnp.sum(v, axis=0), jnp.sum(v * v, axis=0)])

            @pl.when(i == 0)
            def _():
                st_ref[...] = st

            @pl.when(i > 0)
            def _():
                st_ref[...] = st_ref[...] + st

        @pl.when(i == nb)
        def _():
            m = st_ref[0] / float(N)
            var = st_ref[1] / float(N) - m * m
            a = g_ref[...] * lax.rsqrt(var + 1e-5)
            c = b_ref[...] - m * a
            av = vbuf[...] * a + c
            out_ref[0] = av[:, :128]
            out_ref[1] = av[:, 128:]

    clam = lambda i: (jnp.minimum(i, nb - 1), 0)
    clam3 = lambda i: (0, jnp.minimum(i, nb - 1), 0)
    return pl.pallas_call(
        body,
        grid=(nb + 1,),
        in_specs=[
            pl.BlockSpec((bn, F_IN), clam),
            pl.BlockSpec((2, bn, 128), clam3),
            pl.BlockSpec((F_IN, H), lambda i: (0, 0)),
            pl.BlockSpec((1, H), lambda i: (0, 0)),
            pl.BlockSpec((H, H), lambda i: (0, 0)),
            pl.BlockSpec((1, H), lambda i: (0, 0)),
            pl.BlockSpec((1, H), lambda i: (0, 0)),
            pl.BlockSpec((1, H), lambda i: (0, 0)),
        ],
        out_specs=pl.BlockSpec((2, N, 128), lambda i: (0, 0, 0)),
        out_shape=jax.ShapeDtypeStruct((2, N, 128), jnp.float32),
        scratch_shapes=[
            pltpu.VMEM((N, H), jnp.float32),
            pltpu.VMEM((2, H), jnp.float32),
        ],
    )(x, aggP, w1, b1, w2, b2, g, b)


def _tc_final(h0S, agg1S, batch3, w1, b1, w2, b2, g, bb, l1w, l1b, l2w, l2b):
    nb = 10
    bn = N // nb

    def body(h_ref, agg_ref, bt_ref, w1_ref, b1_ref, w2_ref, b2_ref, g_ref,
             bb_ref, l1w_ref, l1b_ref, l2w_ref, l2b_ref, o_ref,
             pooled, cnt, st):
        i = pl.program_id(0)

        @pl.when(i == 0)
        def _():
            pooled[...] = jnp.zeros((G, H), jnp.float32)
            cnt[...] = jnp.zeros((1, G), jnp.float32)
            st[...] = jnp.zeros((2, H), jnp.float32)

        s = jnp.concatenate(
            [h_ref[0] + agg_ref[0], h_ref[1] + agg_ref[1]], axis=1)
        t = jnp.tanh(jnp.dot(s, w1_ref[...], preferred_element_type=jnp.float32)
                     + b1_ref[...])
        u = jnp.dot(t, w2_ref[...], preferred_element_type=jnp.float32) + b2_ref[...]
        v = jnp.tanh(u)

        gids = bt_ref[0, 0]
        oh = (gids[:, None] ==
              lax.broadcasted_iota(jnp.int32, (bn, G), 1)).astype(jnp.float32)
        pooled[...] = pooled[...] + lax.dot_general(
            oh, v, (((0,), (0,)), ((), ())), preferred_element_type=jnp.float32)
        cnt[...] = cnt[...] + jnp.sum(oh, axis=0, keepdims=True)
        st[...] = st[...] + jnp.stack([jnp.sum(v, axis=0), jnp.sum(v * v, axis=0)])

        @pl.when(i == nb - 1)
        def _():
            m = st[0] / float(N)
            var = st[1] / float(N) - m * m
            a = g_ref[...] * lax.rsqrt(var + 1e-5)
            c = bb_ref[...] - m * a
            cc = cnt[...].reshape(G, 1)
            pm = pooled[...] / jnp.maximum(cc, 1.0)
            pb = jnp.where(cc > 0.0, pm * a + c, 0.0)
            o = jnp.dot(jnp.tanh(jnp.dot(pb, l1w_ref[...],
                                         preferred_element_type=jnp.float32)
                                 + l1b_ref[...]),
                        l2w_ref[...], preferred_element_type=jnp.float32)
            o_ref[...] = o + l2b_ref[...]

    return pl.pallas_call(
        body,
        grid=(nb,),
        in_specs=[
            pl.BlockSpec((2, bn, 128), lambda i: (0, i, 0)),
            pl.BlockSpec((2, bn, 128), lambda i: (0, i, 0)),
            pl.BlockSpec((1, 1, bn), lambda i: (i, 0, 0)),
            pl.BlockSpec((H, H), lambda i: (0, 0)),
            pl.BlockSpec((1, H), lambda i: (0, 0)),
            pl.BlockSpec((H, H), lambda i: (0, 0)),
            pl.BlockSpec((1, H), lambda i: (0, 0)),
            pl.BlockSpec((1, H), lambda i: (0, 0)),
            pl.BlockSpec((1, H), lambda i: (0, 0)),
            pl.BlockSpec((H, H), lambda i: (0, 0)),
            pl.BlockSpec((1, H), lambda i: (0, 0)),
            pl.BlockSpec((H, C), lambda i: (0, 0)),
            pl.BlockSpec((1, C), lambda i: (0, 0)),
        ],
        out_specs=pl.BlockSpec((G, C), lambda i: (0, 0)),
        out_shape=jax.ShapeDtypeStruct((G, C), jnp.float32),
        scratch_shapes=[
            pltpu.VMEM((G, H), jnp.float32),
            pltpu.VMEM((1, G), jnp.float32),
            pltpu.VMEM((2, H), jnp.float32),
        ],
    )(h0S, agg1S, batch3, w1, b1, w2, b2, g, bb, l1w, l1b, l2w, l2b)


def kernel(x, edge_index, batch, conv0_w1, conv0_b1, conv0_w2, conv0_b2,
           bn0_g, bn0_b, conv1_w1, conv1_b1, conv1_w2, conv1_b2, bn1_g, bn1_b,
           lin1_w, lin1_b, lin2_w, lin2_b):
    src = edge_index[0].astype(jnp.int32)
    dst = edge_index[1].astype(jnp.int32)
    zeros = jnp.zeros((_ROWS_PER_TILE, 128), jnp.float32)

    nch0 = E // (32 * _K)
    idxA = jnp.stack([src.reshape(32, nch0, _K), dst.reshape(32, nch0, _K)],
                     axis=2)
    agg0P = _sc_segment_sum(x, idxA, zeros, nch0)
    agg0P = agg0P.reshape(2, _NPAD, 128)

    h0S = _tc_conv0bn(
        x, agg0P, conv0_w1, conv0_b1.reshape(1, H), conv0_w2,
        conv0_b2.reshape(1, H), bn0_g.reshape(1, H), bn0_b.reshape(1, H))

    nch1 = E // (16 * _K)
    s3 = src.reshape(1, 16, nch1, _K)
    off = (jnp.arange(2, dtype=jnp.int32) * N).reshape(2, 1, 1, 1)
    srcB = (s3 + off).reshape(32, nch1, _K)
    dstB = jnp.broadcast_to(
        dst.reshape(1, 16, nch1, _K), (2, 16, nch1, _K)).reshape(32, nch1, _K)
    idxB = jnp.stack([srcB, dstB], axis=2)
    agg1S = _sc_segment_sum(h0S.reshape(2 * N, 128), idxB, zeros, nch1)
    agg1S = agg1S.reshape(2, _NPAD, 128)

    o = _tc_final(
        h0S, agg1S, batch.astype(jnp.int32).reshape(10, 1, N // 10),
        conv1_w1, conv1_b1.reshape(1, H), conv1_w2, conv1_b2.reshape(1, H),
        bn1_g.reshape(1, H), bn1_b.reshape(1, H),
        lin1_w, lin1_b.reshape(1, H), lin2_w, lin2_b.reshape(1, C))
    return o


# speedup vs baseline: 1.4634x; 1.0303x over previous
import functools

import jax
import jax.numpy as jnp
from jax import lax
from jax.experimental import pallas as pl
from jax.experimental.pallas import tpu as pltpu
from jax.experimental.pallas import tpu_sc as plsc

N = 10000
E = 320000
F_IN = 128
H = 256
C = 32
G = 64

_K = 80
_NPAD = 10112
_ROWS_PER_TILE = _NPAD // 16


def _sc_segment_sum(table, src, dst, zeros, nchunks, split_features):
    mesh = plsc.VectorSubcoreMesh(
        core_axis_name="c", subcore_axis_name="s", num_cores=2, num_subcores=16)

    @functools.partial(
        pl.kernel,
        out_type=jax.ShapeDtypeStruct((2 * _NPAD, 128), jnp.float32),
        mesh=mesh,
        scratch_types=[
            pltpu.VMEM((6, 1, _K), jnp.int32),
            pltpu.VMEM((6, 1, _K), jnp.int32),
            pltpu.VMEM((4, _K, 128), jnp.float32),
            pltpu.VMEM_SHARED((_NPAD, 128), jnp.float32),
            pltpu.SemaphoreType.DMA,
            pltpu.SemaphoreType.DMA,
            pltpu.SemaphoreType.DMA,
        ],
    )
    def k(table_h, src_h, dst_h, zeros_h, out_h, src_v, dst_v, rows_v, acc,
          gsem, isem, ssem):
        cid = lax.axis_index("c")
        sid = lax.axis_index("s")
        widx = sid if split_features else cid * 16 + sid
        tbl = table_h.at[cid] if split_features else table_h.at[0]
        pltpu.sync_copy(zeros_h, acc.at[pl.ds(sid * _ROWS_PER_TILE, _ROWS_PER_TILE)])
        plsc.subcore_barrier()

        def idx_load(chunk, islot):
            pltpu.async_copy(src_h.at[widx, chunk], src_v.at[islot], isem)
            pltpu.async_copy(dst_h.at[widx, chunk], dst_v.at[islot], isem)

        def idx_wait(chunk, islot):
            pltpu.make_async_copy(src_h.at[widx, chunk], src_v.at[islot],
                                  isem).wait()
            pltpu.make_async_copy(dst_h.at[widx, chunk], dst_v.at[islot],
                                  isem).wait()

        def gather_start(islot, rslot):
            pltpu.async_copy(tbl.at[src_v.at[islot, 0]], rows_v.at[rslot],
                             gsem)

        def gather_wait(islot, rslot):
            pltpu.make_async_copy(tbl.at[src_v.at[islot, 0]],
                                  rows_v.at[rslot], gsem).wait()

        def scatter_start(islot, rslot):
            pltpu.async_copy(rows_v.at[rslot], acc.at[dst_v.at[islot, 0]],
                             ssem, add=True)

        def scatter_wait(islot, rslot):
            pltpu.make_async_copy(rows_v.at[rslot], acc.at[dst_v.at[islot, 0]],
                                  ssem).wait()

        idx_load(0, 0)
        idx_load(jnp.minimum(1, nchunks - 1), 1)
        idx_load(jnp.minimum(2, nchunks - 1), 2)
        idx_wait(0, 0)
        gather_start(0, 0)
        idx_wait(jnp.minimum(1, nchunks - 1), 1)
        gather_start(1, 1)

        def body(j, carry):
            jn2 = jnp.minimum(j + 2, nchunks - 1)
            jn3 = jnp.minimum(j + 3, nchunks - 1)

            @pl.when(j > 1)
            def _():
                scatter_wait((j - 2) % 6, (j + 2) % 4)

            idx_load(jn3, (j + 3) % 6)
            idx_wait(jn2, (j + 2) % 6)
            gather_wait(j % 6, j % 4)
            gather_start((j + 2) % 6, (j + 2) % 4)
            scatter_start(j % 6, j % 4)
            return carry

        lax.fori_loop(0, nchunks, body, 0)
        nc = nchunks
        scatter_wait((nc - 2) % 6, (nc - 2) % 4)
        scatter_wait((nc - 1) % 6, (nc - 1) % 4)
        gather_wait(nc % 6, nc % 4)
        gather_wait((nc + 1) % 6, (nc + 1) % 4)
        idx_wait(nc - 1, (nc + 2) % 6)
        plsc.subcore_barrier()
        base = sid * _ROWS_PER_TILE
        pltpu.sync_copy(
            acc.at[pl.ds(base, _ROWS_PER_TILE)],
            out_h.at[pl.ds(cid * _NPAD + base, _ROWS_PER_TILE)],
        )

    return k(table, src, dst, zeros)


def _tc_conv0bn(x, aggP, w1, b1, w2, b2, g, b):
    nb = 10
    bn = N // nb

    def body(x_ref, agg_ref, w1_ref, b1_ref, w2_ref, b2_ref, g_ref, b_ref,
             out_ref, vbuf, st_ref):
        i = pl.program_id(0)

        @pl.when(i < nb)
        def _():
            s = x_ref[...] + agg_ref[0] + agg_ref[1]
            t = jnp.tanh(jnp.dot(s, w1_ref[...],
                                 preferred_element_type=jnp.float32)
                         + b1_ref[...])
            u = (jnp.dot(t, w2_ref[...], preferred_element_type=jnp.float32)
                 + b2_ref[...])
            v = jnp.tanh(u)
            vbuf[pl.ds(i * bn, bn), :] = v
            st = jnp.stack([jnp.sum(v, axis=0), jnp.sum(v * v, axis=0)])

            @pl.when(i == 0)
            def _():
                st_ref[...] = st

            @pl.when(i > 0)
            def _():
                st_ref[...] = st_ref[...] + st

        @pl.when(i == nb)
        def _():
            m = st_ref[0] / float(N)
            var = st_ref[1] / float(N) - m * m
            a = g_ref[...] * lax.rsqrt(var + 1e-5)
            c = b_ref[...] - m * a
            av = vbuf[...] * a + c
            out_ref[0] = av[:, :128]
            out_ref[1] = av[:, 128:]

    clam = lambda i: (jnp.minimum(i, nb - 1), 0)
    clam3 = lambda i: (0, jnp.minimum(i, nb - 1), 0)
    return pl.pallas_call(
        body,
        grid=(nb + 1,),
        in_specs=[
            pl.BlockSpec((bn, F_IN), clam),
            pl.BlockSpec((2, bn, 128), clam3),
            pl.BlockSpec((F_IN, H), lambda i: (0, 0)),
            pl.BlockSpec((1, H), lambda i: (0, 0)),
            pl.BlockSpec((H, H), lambda i: (0, 0)),
            pl.BlockSpec((1, H), lambda i: (0, 0)),
            pl.BlockSpec((1, H), lambda i: (0, 0)),
            pl.BlockSpec((1, H), lambda i: (0, 0)),
        ],
        out_specs=pl.BlockSpec((2, N, 128), lambda i: (0, 0, 0)),
        out_shape=jax.ShapeDtypeStruct((2, N, 128), jnp.float32),
        scratch_shapes=[
            pltpu.VMEM((N, H), jnp.float32),
            pltpu.VMEM((2, H), jnp.float32),
        ],
    )(x, aggP, w1, b1, w2, b2, g, b)


def _tc_final(h0S, agg1S, batch3, w1, b1, w2, b2, g, bb, l1w, l1b, l2w, l2b):
    nb = 10
    bn = N // nb

    def body(h_ref, agg_ref, bt_ref, w1_ref, b1_ref, w2_ref, b2_ref, g_ref,
             bb_ref, l1w_ref, l1b_ref, l2w_ref, l2b_ref, o_ref,
             pooled, cnt, st):
        i = pl.program_id(0)

        @pl.when(i == 0)
        def _():
            pooled[...] = jnp.zeros((G, H), jnp.float32)
            cnt[...] = jnp.zeros((1, G), jnp.float32)
            st[...] = jnp.zeros((2, H), jnp.float32)

        s = jnp.concatenate(
            [h_ref[0] + agg_ref[0], h_ref[1] + agg_ref[1]], axis=1)
        t = jnp.tanh(jnp.dot(s, w1_ref[...], preferred_element_type=jnp.float32)
                     + b1_ref[...])
        u = jnp.dot(t, w2_ref[...], preferred_element_type=jnp.float32) + b2_ref[...]
        v = jnp.tanh(u)

        gids = bt_ref[0, 0]
        oh = (gids[:, None] ==
              lax.broadcasted_iota(jnp.int32, (bn, G), 1)).astype(jnp.float32)
        pooled[...] = pooled[...] + lax.dot_general(
            oh, v, (((0,), (0,)), ((), ())), preferred_element_type=jnp.float32)
        cnt[...] = cnt[...] + jnp.sum(oh, axis=0, keepdims=True)
        st[...] = st[...] + jnp.stack([jnp.sum(v, axis=0), jnp.sum(v * v, axis=0)])

        @pl.when(i == nb - 1)
        def _():
            m = st[0] / float(N)
            var = st[1] / float(N) - m * m
            a = g_ref[...] * lax.rsqrt(var + 1e-5)
            c = bb_ref[...] - m * a
            cc = cnt[...].reshape(G, 1)
            pm = pooled[...] / jnp.maximum(cc, 1.0)
            pb = jnp.where(cc > 0.0, pm * a + c, 0.0)
            o = jnp.dot(jnp.tanh(jnp.dot(pb, l1w_ref[...],
                                         preferred_element_type=jnp.float32)
                                 + l1b_ref[...]),
                        l2w_ref[...], preferred_element_type=jnp.float32)
            o_ref[...] = o + l2b_ref[...]

    return pl.pallas_call(
        body,
        grid=(nb,),
        in_specs=[
            pl.BlockSpec((2, bn, 128), lambda i: (0, i, 0)),
            pl.BlockSpec((2, bn, 128), lambda i: (0, i, 0)),
            pl.BlockSpec((1, 1, bn), lambda i: (i, 0, 0)),
            pl.BlockSpec((H, H), lambda i: (0, 0)),
            pl.BlockSpec((1, H), lambda i: (0, 0)),
            pl.BlockSpec((H, H), lambda i: (0, 0)),
            pl.BlockSpec((1, H), lambda i: (0, 0)),
            pl.BlockSpec((1, H), lambda i: (0, 0)),
            pl.BlockSpec((1, H), lambda i: (0, 0)),
            pl.BlockSpec((H, H), lambda i: (0, 0)),
            pl.BlockSpec((1, H), lambda i: (0, 0)),
            pl.BlockSpec((H, C), lambda i: (0, 0)),
            pl.BlockSpec((1, C), lambda i: (0, 0)),
        ],
        out_specs=pl.BlockSpec((G, C), lambda i: (0, 0)),
        out_shape=jax.ShapeDtypeStruct((G, C), jnp.float32),
        scratch_shapes=[
            pltpu.VMEM((G, H), jnp.float32),
            pltpu.VMEM((1, G), jnp.float32),
            pltpu.VMEM((2, H), jnp.float32),
        ],
    )(h0S, agg1S, batch3, w1, b1, w2, b2, g, bb, l1w, l1b, l2w, l2b)


def kernel(x, edge_index, batch, conv0_w1, conv0_b1, conv0_w2, conv0_b2,
           bn0_g, bn0_b, conv1_w1, conv1_b1, conv1_w2, conv1_b2, bn1_g, bn1_b,
           lin1_w, lin1_b, lin2_w, lin2_b):
    src = edge_index[0].astype(jnp.int32)
    dst = edge_index[1].astype(jnp.int32)
    zeros = jnp.zeros((_ROWS_PER_TILE, 128), jnp.float32)

    nch0 = E // (32 * _K)
    agg0P = _sc_segment_sum(
        x.reshape(1, N, F_IN), src.reshape(32, nch0, 1, _K),
        dst.reshape(32, nch0, 1, _K), zeros, nch0, split_features=False)
    agg0P = agg0P.reshape(2, _NPAD, 128)

    h0S = _tc_conv0bn(
        x, agg0P, conv0_w1, conv0_b1.reshape(1, H), conv0_w2,
        conv0_b2.reshape(1, H), bn0_g.reshape(1, H), bn0_b.reshape(1, H))

    nch1 = E // (16 * _K)
    agg1S = _sc_segment_sum(
        h0S, src.reshape(16, nch1, 1, _K), dst.reshape(16, nch1, 1, _K),
        zeros, nch1, split_features=True)
    agg1S = agg1S.reshape(2, _NPAD, 128)

    o = _tc_final(
        h0S, agg1S, batch.astype(jnp.int32).reshape(10, 1, N // 10),
        conv1_w1, conv1_b1.reshape(1, H), conv1_w2, conv1_b2.reshape(1, H),
        bn1_g.reshape(1, H), bn1_b.reshape(1, H),
        lin1_w, lin1_b.reshape(1, H), lin2_w, lin2_b.reshape(1, C))
    return o


# speedup vs baseline: 1.4950x; 1.0216x over previous
import functools

import jax
import jax.numpy as jnp
from jax import lax
from jax.experimental import pallas as pl
from jax.experimental.pallas import tpu as pltpu
from jax.experimental.pallas import tpu_sc as plsc

N = 10000
E = 320000
F_IN = 128
H = 256
C = 32
G = 64

_K = 80
_NPAD = 10112
_ROWS_PER_TILE = _NPAD // 16


def _sc_segment_sum(table, src, dst, zeros, nchunks, split_features):
    mesh = plsc.VectorSubcoreMesh(
        core_axis_name="c", subcore_axis_name="s", num_cores=2, num_subcores=16)

    @functools.partial(
        pl.kernel,
        out_type=jax.ShapeDtypeStruct((2 * _NPAD, 128), jnp.float32),
        mesh=mesh,
        scratch_types=[
            pltpu.VMEM((6, 1, _K), jnp.int32),
            pltpu.VMEM((6, 1, _K), jnp.int32),
            pltpu.VMEM((4, _K, 128), jnp.float32),
            pltpu.VMEM_SHARED((_NPAD, 128), jnp.float32),
            pltpu.SemaphoreType.DMA,
            pltpu.SemaphoreType.DMA,
            pltpu.SemaphoreType.DMA,
        ],
    )
    def k(table_h, src_h, dst_h, zeros_h, out_h, src_v, dst_v, rows_v, acc,
          gsem, isem, ssem):
        cid = lax.axis_index("c")
        sid = lax.axis_index("s")
        widx = sid if split_features else cid * 16 + sid
        tbl = table_h.at[cid] if split_features else table_h.at[0]
        zrows = acc.at[pl.ds(sid * _ROWS_PER_TILE, _ROWS_PER_TILE)]
        pltpu.async_copy(zeros_h, zrows, ssem)

        def idx_load(chunk, islot):
            pltpu.async_copy(src_h.at[widx, chunk], src_v.at[islot], isem)
            pltpu.async_copy(dst_h.at[widx, chunk], dst_v.at[islot], isem)

        def idx_wait(chunk, islot):
            pltpu.make_async_copy(src_h.at[widx, chunk], src_v.at[islot],
                                  isem).wait()
            pltpu.make_async_copy(dst_h.at[widx, chunk], dst_v.at[islot],
                                  isem).wait()

        def gather_start(islot, rslot):
            pltpu.async_copy(tbl.at[src_v.at[islot, 0]], rows_v.at[rslot],
                             gsem)

        def gather_wait(islot, rslot):
            pltpu.make_async_copy(tbl.at[src_v.at[islot, 0]],
                                  rows_v.at[rslot], gsem).wait()

        def scatter_start(islot, rslot):
            pltpu.async_copy(rows_v.at[rslot], acc.at[dst_v.at[islot, 0]],
                             ssem, add=True)

        def scatter_wait(islot, rslot):
            pltpu.make_async_copy(rows_v.at[rslot], acc.at[dst_v.at[islot, 0]],
                                  ssem).wait()

        idx_load(0, 0)
        idx_load(jnp.minimum(1, nchunks - 1), 1)
        idx_load(jnp.minimum(2, nchunks - 1), 2)
        idx_wait(0, 0)
        gather_start(0, 0)
        idx_wait(jnp.minimum(1, nchunks - 1), 1)
        gather_start(1, 1)
        pltpu.make_async_copy(zeros_h, zrows, ssem).wait()
        plsc.subcore_barrier()

        def body(j, carry):
            jn2 = jnp.minimum(j + 2, nchunks - 1)
            jn3 = jnp.minimum(j + 3, nchunks - 1)

            @pl.when(j > 1)
            def _():
                scatter_wait((j - 2) % 6, (j + 2) % 4)

            idx_load(jn3, (j + 3) % 6)
            idx_wait(jn2, (j + 2) % 6)
            gather_wait(j % 6, j % 4)
            gather_start((j + 2) % 6, (j + 2) % 4)
            scatter_start(j % 6, j % 4)
            return carry

        lax.fori_loop(0, nchunks, body, 0)
        nc = nchunks
        scatter_wait((nc - 2) % 6, (nc - 2) % 4)
        scatter_wait((nc - 1) % 6, (nc - 1) % 4)
        gather_wait(nc % 6, nc % 4)
        gather_wait((nc + 1) % 6, (nc + 1) % 4)
        idx_wait(nc - 1, (nc + 2) % 6)
        plsc.subcore_barrier()
        base = sid * _ROWS_PER_TILE
        pltpu.sync_copy(
            acc.at[pl.ds(base, _ROWS_PER_TILE)],
            out_h.at[pl.ds(cid * _NPAD + base, _ROWS_PER_TILE)],
        )

    return k(table, src, dst, zeros)


def _tc_conv0bn(x, aggP, w1, b1, w2, b2, g, b):
    nb = 5
    bn = N // nb

    def body(x_ref, agg_ref, w1_ref, b1_ref, w2_ref, b2_ref, g_ref, b_ref,
             out_ref, vbuf, st_ref):
        i = pl.program_id(0)

        @pl.when(i < nb)
        def _():
            s = x_ref[...] + agg_ref[0] + agg_ref[1]
            t = jnp.tanh(jnp.dot(s, w1_ref[...],
                                 preferred_element_type=jnp.float32)
                         + b1_ref[...])
            u = (jnp.dot(t, w2_ref[...], preferred_element_type=jnp.float32)
                 + b2_ref[...])
            v = jnp.tanh(u)
            vbuf[pl.ds(i * bn, bn), :] = v
            st = jnp.stack([jnp.sum(v, axis=0), jnp.sum(v * v, axis=0)])

            @pl.when(i == 0)
            def _():
                st_ref[...] = st

            @pl.when(i > 0)
            def _():
                st_ref[...] = st_ref[...] + st

        @pl.when(i == nb)
        def _():
            m = st_ref[0] / float(N)
            var = st_ref[1] / float(N) - m * m
            a = g_ref[...] * lax.rsqrt(var + 1e-5)
            c = b_ref[...] - m * a
            av = vbuf[...] * a + c
            out_ref[0] = av[:, :128]
            out_ref[1] = av[:, 128:]

    clam = lambda i: (jnp.minimum(i, nb - 1), 0)
    clam3 = lambda i: (0, jnp.minimum(i, nb - 1), 0)
    return pl.pallas_call(
        body,
        grid=(nb + 1,),
        in_specs=[
            pl.BlockSpec((bn, F_IN), clam),
            pl.BlockSpec((2, bn, 128), clam3),
            pl.BlockSpec((F_IN, H), lambda i: (0, 0)),
            pl.BlockSpec((1, H), lambda i: (0, 0)),
            pl.BlockSpec((H, H), lambda i: (0, 0)),
            pl.BlockSpec((1, H), lambda i: (0, 0)),
            pl.BlockSpec((1, H), lambda i: (0, 0)),
            pl.BlockSpec((1, H), lambda i: (0, 0)),
        ],
        out_specs=pl.BlockSpec((2, N, 128), lambda i: (0, 0, 0)),
        out_shape=jax.ShapeDtypeStruct((2, N, 128), jnp.float32),
        scratch_shapes=[
            pltpu.VMEM((N, H), jnp.float32),
            pltpu.VMEM((2, H), jnp.float32),
        ],
    )(x, aggP, w1, b1, w2, b2, g, b)


def _tc_final(h0S, agg1S, batch3, w1, b1, w2, b2, g, bb, l1w, l1b, l2w, l2b):
    nb = 5
    bn = N // nb

    def body(h_ref, agg_ref, bt_ref, w1_ref, b1_ref, w2_ref, b2_ref, g_ref,
             bb_ref, l1w_ref, l1b_ref, l2w_ref, l2b_ref, o_ref,
             pooled, cnt, st):
        i = pl.program_id(0)

        @pl.when(i == 0)
        def _():
            pooled[...] = jnp.zeros((G, H), jnp.float32)
            cnt[...] = jnp.zeros((1, G), jnp.float32)
            st[...] = jnp.zeros((2, H), jnp.float32)

        s = jnp.concatenate(
            [h_ref[0] + agg_ref[0], h_ref[1] + agg_ref[1]], axis=1)
        t = jnp.tanh(jnp.dot(s, w1_ref[...], preferred_element_type=jnp.float32)
                     + b1_ref[...])
        u = jnp.dot(t, w2_ref[...], preferred_element_type=jnp.float32) + b2_ref[...]
        v = jnp.tanh(u)

        gids = bt_ref[0, 0]
        oh = (gids[:, None] ==
              lax.broadcasted_iota(jnp.int32, (bn, G), 1)).astype(jnp.float32)
        pooled[...] = pooled[...] + lax.dot_general(
            oh, v, (((0,), (0,)), ((), ())), preferred_element_type=jnp.float32)
        cnt[...] = cnt[...] + jnp.sum(oh, axis=0, keepdims=True)
        st[...] = st[...] + jnp.stack([jnp.sum(v, axis=0), jnp.sum(v * v, axis=0)])

        @pl.when(i == nb - 1)
        def _():
            m = st[0] / float(N)
            var = st[1] / float(N) - m * m
            a = g_ref[...] * lax.rsqrt(var + 1e-5)
            c = bb_ref[...] - m * a
            cc = cnt[...].reshape(G, 1)
            pm = pooled[...] / jnp.maximum(cc, 1.0)
            pb = jnp.where(cc > 0.0, pm * a + c, 0.0)
            o = jnp.dot(jnp.tanh(jnp.dot(pb, l1w_ref[...],
                                         preferred_element_type=jnp.float32)
                                 + l1b_ref[...]),
                        l2w_ref[...], preferred_element_type=jnp.float32)
            o_ref[...] = o + l2b_ref[...]

    return pl.pallas_call(
        body,
        grid=(nb,),
        in_specs=[
            pl.BlockSpec((2, bn, 128), lambda i: (0, i, 0)),
            pl.BlockSpec((2, bn, 128), lambda i: (0, i, 0)),
            pl.BlockSpec((1, 1, bn), lambda i: (i, 0, 0)),
            pl.BlockSpec((H, H), lambda i: (0, 0)),
            pl.BlockSpec((1, H), lambda i: (0, 0)),
            pl.BlockSpec((H, H), lambda i: (0, 0)),
            pl.BlockSpec((1, H), lambda i: (0, 0)),
            pl.BlockSpec((1, H), lambda i: (0, 0)),
            pl.BlockSpec((1, H), lambda i: (0, 0)),
            pl.BlockSpec((H, H), lambda i: (0, 0)),
            pl.BlockSpec((1, H), lambda i: (0, 0)),
            pl.BlockSpec((H, C), lambda i: (0, 0)),
            pl.BlockSpec((1, C), lambda i: (0, 0)),
        ],
        out_specs=pl.BlockSpec((G, C), lambda i: (0, 0)),
        out_shape=jax.ShapeDtypeStruct((G, C), jnp.float32),
        scratch_shapes=[
            pltpu.VMEM((G, H), jnp.float32),
            pltpu.VMEM((1, G), jnp.float32),
            pltpu.VMEM((2, H), jnp.float32),
        ],
    )(h0S, agg1S, batch3, w1, b1, w2, b2, g, bb, l1w, l1b, l2w, l2b)


def kernel(x, edge_index, batch, conv0_w1, conv0_b1, conv0_w2, conv0_b2,
           bn0_g, bn0_b, conv1_w1, conv1_b1, conv1_w2, conv1_b2, bn1_g, bn1_b,
           lin1_w, lin1_b, lin2_w, lin2_b):
    src = edge_index[0].astype(jnp.int32)
    dst = edge_index[1].astype(jnp.int32)
    zeros = jnp.zeros((_ROWS_PER_TILE, 128), jnp.float32)

    nch0 = E // (32 * _K)
    agg0P = _sc_segment_sum(
        x.reshape(1, N, F_IN), src.reshape(32, nch0, 1, _K),
        dst.reshape(32, nch0, 1, _K), zeros, nch0, split_features=False)
    agg0P = agg0P.reshape(2, _NPAD, 128)

    h0S = _tc_conv0bn(
        x, agg0P, conv0_w1, conv0_b1.reshape(1, H), conv0_w2,
        conv0_b2.reshape(1, H), bn0_g.reshape(1, H), bn0_b.reshape(1, H))

    nch1 = E // (16 * _K)
    agg1S = _sc_segment_sum(
        h0S, src.reshape(16, nch1, 1, _K), dst.reshape(16, nch1, 1, _K),
        zeros, nch1, split_features=True)
    agg1S = agg1S.reshape(2, _NPAD, 128)

    o = _tc_final(
        h0S, agg1S, batch.astype(jnp.int32).reshape(5, 1, N // 5),
        conv1_w1, conv1_b1.reshape(1, H), conv1_w2, conv1_b2.reshape(1, H),
        bn1_g.reshape(1, H), bn1_b.reshape(1, H),
        lin1_w, lin1_b.reshape(1, H), lin2_w, lin2_b.reshape(1, C))
    return o
